# Initial kernel scaffold; baseline (speedup 1.0000x reference)
import jax
import jax.numpy as jnp
from jax.experimental import pallas as pl


def kernel(x, edge_index, batch, W_rel0, b_rel0, W_root0, W_rel1, b_rel1, W_root1, W_rel2, b_rel2, W_root2, W_out, b_out):
    raise NotImplementedError("write your pallas kernel here")


# speedup vs baseline: 6.9767x; 6.9767x over previous
import functools

import jax
import jax.numpy as jnp
from jax import lax
from jax.experimental import pallas as pl
from jax.experimental.pallas import tpu as pltpu
from jax.experimental.pallas import tpu_sc as plsc

_N = 10000
_E = 320000
_H = 128
_G = 64
_C = 10

_NC = 2
_NS = 16
_NW = _NC * _NS
_EPW = _E // _NW
_CHUNK = 80
_NCHUNK = _EPW // _CHUNK
_NPAD = 10240
_RPT = _NPAD // _NS


def _seg_sum_body(h_hbm, src_hbm, dst_hbm, zero_hbm, out_hbm,
                  src_v, dst_v, rows_v, agg_sh, gsem):
    c = lax.axis_index("c")
    s = lax.axis_index("s")
    wid = s * _NC + c
    r0 = s * _RPT
    pltpu.sync_copy(zero_hbm.at[pl.ds(r0, _RPT)], agg_sh.at[pl.ds(r0, _RPT)])
    pltpu.sync_copy(src_hbm.at[wid], src_v)
    pltpu.sync_copy(dst_hbm.at[wid], dst_v)
    plsc.subcore_barrier()

    def body(i, carry):
        pltpu.async_copy(h_hbm.at[src_v.at[i]], rows_v, gsem).wait()
        pltpu.sync_copy(rows_v, agg_sh.at[dst_v.at[i]], add=True)
        return carry

    lax.fori_loop(0, _NCHUNK, body, 0)
    plsc.subcore_barrier()
    pltpu.sync_copy(agg_sh.at[pl.ds(r0, _RPT)],
                    out_hbm.at[c, pl.ds(r0, _RPT)])


def _make_seg_sum():
    mesh = plsc.VectorSubcoreMesh(core_axis_name="c", subcore_axis_name="s")
    return pl.kernel(
        _seg_sum_body,
        mesh=mesh,
        out_type=jax.ShapeDtypeStruct((_NC, _NPAD, _H), jnp.float32),
        scratch_types=[
            pltpu.VMEM((_NCHUNK, _CHUNK), jnp.int32),
            pltpu.VMEM((_NCHUNK, _CHUNK), jnp.int32),
            pltpu.VMEM((_CHUNK, _H), jnp.float32),
            pltpu.VMEM_SHARED((_NPAD, _H), jnp.float32),
            pltpu.SemaphoreType.DMA,
        ],
    )


_BN = 400


def _dense_block(agg_ref, h_ref, wr_ref, br_ref, wt_ref, out_ref):
    a = agg_ref[0] + agg_ref[1]
    acc = jnp.dot(a, wr_ref[...], preferred_element_type=jnp.float32)
    acc += jnp.dot(h_ref[...], wt_ref[...], preferred_element_type=jnp.float32)
    acc += br_ref[...]
    out_ref[...] = jnp.maximum(acc, 0.0)


def _dense(agg2, h, wr, br, wt):
    return pl.pallas_call(
        _dense_block,
        grid=(_N // _BN,),
        in_specs=[
            pl.BlockSpec((_NC, _BN, _H), lambda i: (0, i, 0)),
            pl.BlockSpec((_BN, _H), lambda i: (i, 0)),
            pl.BlockSpec((_H, _H), lambda i: (0, 0)),
            pl.BlockSpec((1, _H), lambda i: (0, 0)),
            pl.BlockSpec((_H, _H), lambda i: (0, 0)),
        ],
        out_specs=pl.BlockSpec((_BN, _H), lambda i: (i, 0)),
        out_shape=jax.ShapeDtypeStruct((_N, _H), jnp.float32),
    )(agg2, h, wr, br.reshape(1, _H), wt)


_BP = 400


def _pool_block(h_ref, b_ref, wo_ref, bo_ref, out_ref, psum, cnt):
    i = pl.program_id(0)

    @pl.when(i == 0)
    def _():
        psum[...] = jnp.zeros_like(psum)
        cnt[...] = jnp.zeros_like(cnt)

    bvals = b_ref[0, 0, :]
    gids = lax.broadcasted_iota(jnp.int32, (_G, _BP), 0)
    onehot = (bvals[None, :] == gids).astype(jnp.float32)
    psum[...] += jnp.dot(onehot, h_ref[...], preferred_element_type=jnp.float32)
    cnt[...] += jnp.sum(onehot, axis=1, keepdims=True)

    @pl.when(i == pl.num_programs(0) - 1)
    def _():
        pooled = psum[...] / jnp.maximum(cnt[...], 1.0)
        out_ref[...] = (jnp.dot(pooled, wo_ref[...],
                                preferred_element_type=jnp.float32)
                        + bo_ref[...])


def _pool(h, batch, wo, bo):
    return pl.pallas_call(
        _pool_block,
        grid=(_N // _BP,),
        in_specs=[
            pl.BlockSpec((_BP, _H), lambda i: (i, 0)),
            pl.BlockSpec((1, 1, _BP), lambda i: (i, 0, 0)),
            pl.BlockSpec((_H, _C), lambda i: (0, 0)),
            pl.BlockSpec((1, _C), lambda i: (0, 0)),
        ],
        out_specs=pl.BlockSpec((_G, _C), lambda i: (0, 0)),
        out_shape=jax.ShapeDtypeStruct((_G, _C), jnp.float32),
        scratch_shapes=[
            pltpu.VMEM((_G, _H), jnp.float32),
            pltpu.VMEM((_G, 1), jnp.float32),
        ],
    )(h, batch.reshape(_N // _BP, 1, _BP), wo, bo.reshape(1, _C))


def kernel(x, edge_index, batch, W_rel0, b_rel0, W_root0, W_rel1, b_rel1,
           W_root1, W_rel2, b_rel2, W_root2, W_out, b_out):
    src = edge_index[0].reshape(_NW, _NCHUNK, _CHUNK)
    dst = edge_index[1].reshape(_NW, _NCHUNK, _CHUNK)
    zeros = jnp.zeros((_NPAD, _H), jnp.float32)
    seg_sum = _make_seg_sum()
    h = x
    for wr, br, wt in ((W_rel0, b_rel0, W_root0),
                       (W_rel1, b_rel1, W_root1),
                       (W_rel2, b_rel2, W_root2)):
        agg2 = seg_sum(h, src, dst, zeros)
        h = _dense(agg2, h, wr, br, wt)
    return _pool(h, batch, W_out, b_out)
